# baseline (speedup 1.0000x reference)
import functools

import jax
import jax.numpy as jnp
from jax import lax
from jax.experimental import pallas as pl
from jax.experimental.pallas import tpu as pltpu
from jax.experimental.pallas import tpu_sc as plsc

BATCH = 16384
EMB_K = 64
NROWS = 100000
NUM_CORES = 2
NUM_SUBCORES = 16
NW = NUM_CORES * NUM_SUBCORES
B_PER_W = BATCH // NW
NBANDS = EMB_K // 8


PP_SUB = 500
PP_INNER = 8
PP_BLK = PP_SUB * PP_INNER
PP_GRID = NROWS // PP_BLK


def _pack_body(tbl_ref, out_ref):
    rows = lax.broadcasted_iota(jnp.int32, (PP_SUB // 2, PP_SUB), 0)
    cols = lax.broadcasted_iota(jnp.int32, (PP_SUB // 2, PP_SUB), 1)
    p_even = (cols == 2 * rows).astype(jnp.float32)
    p_odd = (cols == 2 * rows + 1).astype(jnp.float32)
    for j in range(PP_INNER):
        blk = tbl_ref[pl.ds(PP_SUB * j, PP_SUB), :]
        e = jnp.dot(p_even, blk, preferred_element_type=jnp.float32,
                    precision=lax.Precision.HIGHEST)
        o = jnp.dot(p_odd, blk, preferred_element_type=jnp.float32,
                    precision=lax.Precision.HIGHEST)
        out_ref[pl.ds(PP_SUB // 2 * j, PP_SUB // 2), :] = (
            jnp.concatenate([e, o], axis=1))


def _pair_pack(tbl):
    return pl.pallas_call(
        _pack_body,
        grid=(PP_GRID,),
        in_specs=[pl.BlockSpec((PP_BLK, EMB_K), lambda i: (i, 0))],
        out_specs=pl.BlockSpec((PP_BLK // 2, 128), lambda i: (i, 0)),
        out_shape=jax.ShapeDtypeStruct((PP_GRID * PP_BLK // 2, 128),
                                       jnp.float32),
    )(tbl)


def _sc_gather_body(pidx_hbm, pair_hbm, out_hbm, idx_v, rows_v, sem):
    wid = lax.axis_index("s") * NUM_CORES + lax.axis_index("c")
    base = wid * B_PER_W
    pltpu.sync_copy(pidx_hbm.at[pl.ds(base, B_PER_W)], idx_v)
    pltpu.async_copy(pair_hbm.at[idx_v], rows_v, sem).wait()
    pltpu.sync_copy(rows_v, out_hbm.at[pl.ds(base, B_PER_W)])


@functools.cache
def _sc_gather():
    return pl.kernel(
        _sc_gather_body,
        mesh=plsc.VectorSubcoreMesh(
            core_axis_name="c", subcore_axis_name="s",
            num_cores=NUM_CORES, num_subcores=NUM_SUBCORES),
        out_type=jax.ShapeDtypeStruct((BATCH, 128), jnp.float32),
        scratch_types=[
            pltpu.VMEM((B_PER_W,), jnp.int32),
            pltpu.VMEM((B_PER_W, 128), jnp.float32),
            pltpu.SemaphoreType.DMA,
        ],
        compiler_params=pltpu.CompilerParams(
            use_tc_tiling_on_sc=False, needs_layout_passes=False),
    )


MLP_BLK = 1024


def _mlp_body(up_ref, vp_ref, paru_ref, parv_ref, w1u_ref, w1v_ref,
              b1_ref, w2_ref, out_ref, u4_ref, v4_ref):
    paru = paru_ref[...]
    parv = parv_ref[...]
    up = up_ref[...]
    vp = vp_ref[...]
    u = up[:, :EMB_K] * (1.0 - paru) + up[:, EMB_K:] * paru
    v = vp[:, :EMB_K] * (1.0 - parv) + vp[:, EMB_K:] * parv
    h = (lax.dot_general(u, w1u_ref[...], (((1,), (1,)), ((), ())),
                         preferred_element_type=jnp.float32)
         + lax.dot_general(v, w1v_ref[...], (((1,), (1,)), ((), ())),
                           preferred_element_type=jnp.float32)
         + b1_ref[...])
    h = jnp.maximum(h, 0.0)
    logit = jnp.sum(h * w2_ref[...], axis=1)
    out_ref[0, :] = jax.nn.sigmoid(logit)
    ut = u.T
    vt = v.T
    for tt in range(MLP_BLK // 128):
        u4_ref[:, tt] = ut[:, 128 * tt:128 * (tt + 1)].reshape(NBANDS, 8, 128)
        v4_ref[:, tt] = vt[:, 128 * tt:128 * (tt + 1)].reshape(NBANDS, 8, 128)


def _mlp(up, vp, paru, parv, w1u, w1v, b1, w2):
    grid = (BATCH // MLP_BLK,)
    return pl.pallas_call(
        _mlp_body,
        grid=grid,
        in_specs=[
            pl.BlockSpec((MLP_BLK, 128), lambda i: (i, 0)),
            pl.BlockSpec((MLP_BLK, 128), lambda i: (i, 0)),
            pl.BlockSpec((MLP_BLK, 1), lambda i: (i, 0)),
            pl.BlockSpec((MLP_BLK, 1), lambda i: (i, 0)),
            pl.BlockSpec((EMB_K, EMB_K), lambda i: (0, 0)),
            pl.BlockSpec((EMB_K, EMB_K), lambda i: (0, 0)),
            pl.BlockSpec((1, EMB_K), lambda i: (0, 0)),
            pl.BlockSpec((1, EMB_K), lambda i: (0, 0)),
        ],
        out_specs=[
            pl.BlockSpec((1, MLP_BLK), lambda i: (0, i)),
            pl.BlockSpec((NBANDS, MLP_BLK // 128, 8, 128),
                         lambda i: (0, i, 0, 0)),
            pl.BlockSpec((NBANDS, MLP_BLK // 128, 8, 128),
                         lambda i: (0, i, 0, 0)),
        ],
        out_shape=[
            jax.ShapeDtypeStruct((1, BATCH), jnp.float32),
            jax.ShapeDtypeStruct((NBANDS, BATCH // 128, 8, 128), jnp.float32),
            jax.ShapeDtypeStruct((NBANDS, BATCH // 128, 8, 128), jnp.float32),
        ],
    )(up, vp, paru, parv, w1u, w1v, b1, w2)


def kernel(x, W_table, H_table, W1, b1, W2):
    u_idx = x[:, 0]
    v_idx = x[:, 1]
    u_pidx = lax.shift_right_logical(u_idx, 1)
    v_pidx = lax.shift_right_logical(v_idx, 1)
    paru = jnp.bitwise_and(u_idx, 1).astype(jnp.float32).reshape(BATCH, 1)
    parv = jnp.bitwise_and(v_idx, 1).astype(jnp.float32).reshape(BATCH, 1)
    w_pair = _pair_pack(W_table)
    h_pair = _pair_pack(H_table)
    g = _sc_gather()
    up = g(u_pidx, w_pair)
    vp = g(v_pidx, h_pair)
    w1u = W1[:, :EMB_K]
    w1v = W1[:, EMB_K:]
    out2d, u4, v4 = _mlp(up, vp, paru, parv, w1u, w1v,
                         b1.reshape(1, EMB_K), W2)
    u_emb = u4.transpose(0, 2, 1, 3).reshape(EMB_K, BATCH).T
    v_emb = v4.transpose(0, 2, 1, 3).reshape(EMB_K, BATCH).T
    return (out2d.reshape(BATCH), u_emb, v_emb)


# speedup vs baseline: 2.5256x; 2.5256x over previous
import functools

import jax
import jax.numpy as jnp
from jax import lax
from jax.experimental import pallas as pl
from jax.experimental.pallas import tpu as pltpu
from jax.experimental.pallas import tpu_sc as plsc

BATCH = 16384
EMB_K = 64
NROWS = 100000
NUM_CORES = 2
NUM_SUBCORES = 16
NW = NUM_CORES * NUM_SUBCORES
B_PER_W = BATCH // NW
NBANDS = EMB_K // 8


PP_BLK = 4000
PP_GRID = NROWS // PP_BLK


def _pack_body(tbl_ref, out_ref):
    out_ref[:, :EMB_K] = tbl_ref[...]
    out_ref[:, EMB_K:] = jnp.zeros((PP_BLK, 128 - EMB_K), jnp.float32)


def _widen(tbl):
    return pl.pallas_call(
        _pack_body,
        grid=(PP_GRID,),
        in_specs=[pl.BlockSpec((PP_BLK, EMB_K), lambda i: (i, 0))],
        out_specs=pl.BlockSpec((PP_BLK, 128), lambda i: (i, 0)),
        out_shape=jax.ShapeDtypeStruct((NROWS, 128), jnp.float32),
    )(tbl)


def _sc_gather_body(pidx_hbm, pair_hbm, out_hbm, idx_v, rows_v, sem):
    wid = lax.axis_index("s") * NUM_CORES + lax.axis_index("c")
    base = wid * B_PER_W
    pltpu.sync_copy(pidx_hbm.at[pl.ds(base, B_PER_W)], idx_v)
    pltpu.async_copy(pair_hbm.at[idx_v], rows_v, sem).wait()
    pltpu.sync_copy(rows_v, out_hbm.at[pl.ds(base, B_PER_W)])


@functools.cache
def _sc_gather():
    return pl.kernel(
        _sc_gather_body,
        mesh=plsc.VectorSubcoreMesh(
            core_axis_name="c", subcore_axis_name="s",
            num_cores=NUM_CORES, num_subcores=NUM_SUBCORES),
        out_type=jax.ShapeDtypeStruct((BATCH, 128), jnp.float32),
        scratch_types=[
            pltpu.VMEM((B_PER_W,), jnp.int32),
            pltpu.VMEM((B_PER_W, 128), jnp.float32),
            pltpu.SemaphoreType.DMA,
        ],
        compiler_params=pltpu.CompilerParams(
            use_tc_tiling_on_sc=False, needs_layout_passes=False),
    )


MLP_BLK = 1024


def _mlp_body(up_ref, vp_ref, w1u_ref, w1v_ref,
              b1_ref, w2_ref, out_ref, u4_ref, v4_ref):
    u = up_ref[:, :EMB_K]
    v = vp_ref[:, :EMB_K]
    h = (lax.dot_general(u, w1u_ref[...], (((1,), (1,)), ((), ())),
                         preferred_element_type=jnp.float32)
         + lax.dot_general(v, w1v_ref[...], (((1,), (1,)), ((), ())),
                           preferred_element_type=jnp.float32)
         + b1_ref[...])
    h = jnp.maximum(h, 0.0)
    logit = jnp.sum(h * w2_ref[...], axis=1)
    out_ref[0, :] = jax.nn.sigmoid(logit)
    ut = u.T
    vt = v.T
    for tt in range(MLP_BLK // 128):
        u4_ref[:, tt] = ut[:, 128 * tt:128 * (tt + 1)].reshape(NBANDS, 8, 128)
        v4_ref[:, tt] = vt[:, 128 * tt:128 * (tt + 1)].reshape(NBANDS, 8, 128)


def _mlp(up, vp, w1u, w1v, b1, w2):
    grid = (BATCH // MLP_BLK,)
    return pl.pallas_call(
        _mlp_body,
        grid=grid,
        in_specs=[
            pl.BlockSpec((MLP_BLK, 128), lambda i: (i, 0)),
            pl.BlockSpec((MLP_BLK, 128), lambda i: (i, 0)),
            pl.BlockSpec((EMB_K, EMB_K), lambda i: (0, 0)),
            pl.BlockSpec((EMB_K, EMB_K), lambda i: (0, 0)),
            pl.BlockSpec((1, EMB_K), lambda i: (0, 0)),
            pl.BlockSpec((1, EMB_K), lambda i: (0, 0)),
        ],
        out_specs=[
            pl.BlockSpec((1, MLP_BLK), lambda i: (0, i)),
            pl.BlockSpec((NBANDS, MLP_BLK // 128, 8, 128),
                         lambda i: (0, i, 0, 0)),
            pl.BlockSpec((NBANDS, MLP_BLK // 128, 8, 128),
                         lambda i: (0, i, 0, 0)),
        ],
        out_shape=[
            jax.ShapeDtypeStruct((1, BATCH), jnp.float32),
            jax.ShapeDtypeStruct((NBANDS, BATCH // 128, 8, 128), jnp.float32),
            jax.ShapeDtypeStruct((NBANDS, BATCH // 128, 8, 128), jnp.float32),
        ],
    )(up, vp, w1u, w1v, b1, w2)


def kernel(x, W_table, H_table, W1, b1, W2):
    u_idx = x[:, 0]
    v_idx = x[:, 1]
    w_wide = _widen(W_table)
    h_wide = _widen(H_table)
    g = _sc_gather()
    up = g(u_idx, w_wide)
    vp = g(v_idx, h_wide)
    w1u = W1[:, :EMB_K]
    w1v = W1[:, EMB_K:]
    out2d, u4, v4 = _mlp(up, vp, w1u, w1v, b1.reshape(1, EMB_K), W2)
    u_emb = u4.transpose(0, 2, 1, 3).reshape(EMB_K, BATCH).T
    v_emb = v4.transpose(0, 2, 1, 3).reshape(EMB_K, BATCH).T
    return (out2d.reshape(BATCH), u_emb, v_emb)


# speedup vs baseline: 3.2019x; 1.2678x over previous
import functools

import jax
import jax.numpy as jnp
from jax import lax
from jax.experimental import pallas as pl
from jax.experimental.pallas import tpu as pltpu
from jax.experimental.pallas import tpu_sc as plsc

BATCH = 16384
EMB_K = 64
NUM_CORES = 2
NUM_SUBCORES = 16
NW = NUM_CORES * NUM_SUBCORES
B_PER_W = BATCH // NW
NBANDS = EMB_K // 8
NPAIR = BATCH // 2


def _sc_gather_body(idx_hbm, tbl_hbm, pair_out, idx_v, rows_v, sem):
    wid = lax.axis_index("s") * NUM_CORES + lax.axis_index("c")
    base = wid * B_PER_W
    pltpu.sync_copy(idx_hbm.at[pl.ds(base, B_PER_W)], idx_v)
    cp = pltpu.async_copy(tbl_hbm.at[idx_v], rows_v, sem)
    row0 = 512 * (wid // 2)
    col0 = EMB_K * (wid % 2)
    cp.wait()
    pltpu.sync_copy(rows_v, pair_out.at[pl.ds(row0, B_PER_W), pl.ds(col0, EMB_K)])


@functools.cache
def _sc_gather():
    return pl.kernel(
        _sc_gather_body,
        mesh=plsc.VectorSubcoreMesh(
            core_axis_name="c", subcore_axis_name="s",
            num_cores=NUM_CORES, num_subcores=NUM_SUBCORES),
        out_type=jax.ShapeDtypeStruct((NPAIR, 128), jnp.float32),
        scratch_types=[
            pltpu.VMEM((B_PER_W,), jnp.int32),
            pltpu.VMEM((B_PER_W, EMB_K), jnp.float32),
            pltpu.SemaphoreType.DMA,
        ],
        compiler_params=pltpu.CompilerParams(
            use_tc_tiling_on_sc=False, needs_layout_passes=False),
    )


PAIR_BLK = 1024
LT_HALF = PAIR_BLK // 128


def _mlp_body(u2_ref, v2_ref, w1u_ref, w1v_ref, b1_ref, w2_ref,
              out_ref, u4_ref, v4_ref):
    w1u = w1u_ref[...]
    w1v = w1v_ref[...]
    b1 = b1_ref[...]
    w2 = w2_ref[...]
    up = u2_ref[...]
    vp = v2_ref[...]
    for half in range(2):
        u = up[:, EMB_K * half:EMB_K * (half + 1)]
        v = vp[:, EMB_K * half:EMB_K * (half + 1)]
        h = (lax.dot_general(u, w1u, (((1,), (1,)), ((), ())),
                             preferred_element_type=jnp.float32)
             + lax.dot_general(v, w1v, (((1,), (1,)), ((), ())),
                               preferred_element_type=jnp.float32)
             + b1)
        h = jnp.maximum(h, 0.0)
        logit = jnp.sum(h * w2, axis=1)
        sig = jax.nn.sigmoid(logit)
        out_ref[0, pl.ds(512 * half, 512)] = sig[:512]
        out_ref[0, pl.ds(1024 + 512 * half, 512)] = sig[512:]
        ut = u.T
        vt = v.T
        for tt in range(LT_HALF):
            t = 8 * (tt // 4) + 4 * half + (tt % 4)
            u4_ref[:, t] = ut[:, 128 * tt:128 * (tt + 1)].reshape(NBANDS, 8, 128)
            v4_ref[:, t] = vt[:, 128 * tt:128 * (tt + 1)].reshape(NBANDS, 8, 128)


def _mlp(u2, v2, w1u, w1v, b1, w2):
    grid = (NPAIR // PAIR_BLK,)
    return pl.pallas_call(
        _mlp_body,
        grid=grid,
        in_specs=[
            pl.BlockSpec((PAIR_BLK, 128), lambda i: (i, 0)),
            pl.BlockSpec((PAIR_BLK, 128), lambda i: (i, 0)),
            pl.BlockSpec((EMB_K, EMB_K), lambda i: (0, 0)),
            pl.BlockSpec((EMB_K, EMB_K), lambda i: (0, 0)),
            pl.BlockSpec((1, EMB_K), lambda i: (0, 0)),
            pl.BlockSpec((1, EMB_K), lambda i: (0, 0)),
        ],
        out_specs=[
            pl.BlockSpec((1, 2 * PAIR_BLK), lambda i: (0, i)),
            pl.BlockSpec((NBANDS, 2 * LT_HALF, 8, 128), lambda i: (0, i, 0, 0)),
            pl.BlockSpec((NBANDS, 2 * LT_HALF, 8, 128), lambda i: (0, i, 0, 0)),
        ],
        out_shape=[
            jax.ShapeDtypeStruct((1, BATCH), jnp.float32),
            jax.ShapeDtypeStruct((NBANDS, BATCH // 128, 8, 128), jnp.float32),
            jax.ShapeDtypeStruct((NBANDS, BATCH // 128, 8, 128), jnp.float32),
        ],
    )(u2, v2, w1u, w1v, b1, w2)


def kernel(x, W_table, H_table, W1, b1, W2):
    u_idx = x[:, 0]
    v_idx = x[:, 1]
    g = _sc_gather()
    u2 = g(u_idx, W_table)
    v2 = g(v_idx, H_table)
    w1u = W1[:, :EMB_K]
    w1v = W1[:, EMB_K:]
    out2d, u4, v4 = _mlp(u2, v2, w1u, w1v, b1.reshape(1, EMB_K), W2)
    u_emb = u4.transpose(0, 2, 1, 3).reshape(EMB_K, BATCH).T
    v_emb = v4.transpose(0, 2, 1, 3).reshape(EMB_K, BATCH).T
    return (out2d.reshape(BATCH), u_emb, v_emb)
